# baseline (speedup 1.0000x reference)
import functools

import jax
import jax.numpy as jnp
from jax import lax
from jax.experimental import pallas as pl
from jax.experimental.pallas import tpu as pltpu
from jax.experimental.pallas import tpu_sc as plsc

_B = 16384
_EMB = 64
_NW = 32
_BPW = _B // _NW
_NCH = 4
_CH = 128

_f32 = jnp.float32


_NBLK = 100000 // 128
_TAIL = 100000 - _NBLK * 128
_BLK_PER_TILE = 49


def _sc_transpose(uembT, bembT, utail, btail):
  mesh = plsc.VectorSubcoreMesh(
      core_axis_name="c", subcore_axis_name="s", num_cores=2, num_subcores=16)

  @functools.partial(
      pl.kernel,
      out_type=(
          jax.ShapeDtypeStruct((100000, _EMB), _f32),
          jax.ShapeDtypeStruct((100000, _EMB), _f32),
      ),
      mesh=mesh,
      compiler_params=pltpu.CompilerParams(
          use_tc_tiling_on_sc=True, needs_layout_passes=False),
      scratch_types=[
          pltpu.VMEM((_EMB, 128), _f32),
          pltpu.VMEM((_EMB, 128), _f32),
          pltpu.VMEM((128, _EMB), _f32),
          pltpu.VMEM((128, _EMB), _f32),
          pltpu.SemaphoreType.DMA,
          pltpu.SemaphoreType.DMA,
          pltpu.SemaphoreType.DMA,
      ],
  )
  def tr_k(uembT_h, bembT_h, utail_h, btail_h, uscr_o, bscr_o,
           vin0, vin1, vout0, vout1, sem0, sem1, sem_o):
    core = lax.axis_index("c")
    tid = lax.axis_index("s")
    vins, sems = (vin0, vin1), (sem0, sem1)
    vouts = (vout0, vout1)
    lanes = lax.iota(jnp.int32, 16)

    def work(src_h, dst_h, tail_h):
      def fetch(n, buf):
        j = tid * _BLK_PER_TILE + n
        @pl.when(j < _NBLK)
        def _():
          pltpu.async_copy(
              src_h.at[:, pl.ds(j * 128, 128)], vins[buf], sems[buf])

      def transpose_block(vin, vout, rows):
        def body(r, _):
          rv = jnp.full((16,), r, jnp.int32)
          for k in range(_EMB // 16):
            vals = plsc.load_gather(vin, [lanes + (k * 16), rv])
            vout[r, pl.ds(k * 16, 16)] = vals
          return 0
        lax.fori_loop(0, rows, body, 0)

      fetch(0, 0)
      for n in range(_BLK_PER_TILE):
        buf = n % 2
        j = tid * _BLK_PER_TILE + n
        @pl.when(j < _NBLK)
        def _(n=n, buf=buf, j=j):
          pltpu.make_async_copy(
              src_h.at[:, pl.ds(0, 128)], vins[buf], sems[buf]).wait()
          if n + 1 < _BLK_PER_TILE:
            fetch(n + 1, 1 - buf)
          transpose_block(vins[buf], vouts[buf], 128)
          pltpu.sync_copy(vouts[buf], dst_h.at[pl.ds(j * 128, 128)])

      @pl.when(tid == 0)
      def _():
        tail_copies = []
        for i in range(_TAIL):
          tail_copies.append(
              pltpu.async_copy(tail_h.at[i], vout0.at[i], sem_o))
        for c in tail_copies:
          c.wait()
        pltpu.sync_copy(vout0.at[pl.ds(0, _TAIL)],
                        dst_h.at[pl.ds(_NBLK * 128, _TAIL)])

    @pl.when(core == 0)
    def _():
      work(uembT_h, uscr_o, utail_h)

    @pl.when(core == 1)
    def _():
      work(bembT_h, bscr_o, btail_h)

  return tr_k(uembT, bembT, utail, btail)


def _sc_main(uidx, bidx, uemb, bemb, ubt1, bbt1):
  mesh = plsc.VectorSubcoreMesh(
      core_axis_name="c", subcore_axis_name="s", num_cores=2, num_subcores=16)

  @functools.partial(
      pl.kernel,
      out_type=(
          jax.ShapeDtypeStruct((_NW, 128), _f32),
          jax.ShapeDtypeStruct((128, 128), _f32),
          jax.ShapeDtypeStruct((128, 128), _f32),
      ),
      mesh=mesh,
      compiler_params=pltpu.CompilerParams(use_tc_tiling_on_sc=True),
      scratch_types=[
          pltpu.VMEM((_NCH, _CH), jnp.int32),
          pltpu.VMEM((_NCH, _CH), jnp.int32),
          pltpu.VMEM((_CH, _EMB), _f32),
          pltpu.VMEM((_CH, _EMB), _f32),
          pltpu.VMEM((_CH, _EMB), _f32),
          pltpu.VMEM((_CH, _EMB), _f32),
          pltpu.VMEM((_NCH, _CH), _f32),
          pltpu.VMEM((_NCH, _CH), _f32),
          pltpu.VMEM((128,), _f32),
          pltpu.SemaphoreType.DMA,
          pltpu.SemaphoreType.DMA,
          pltpu.SemaphoreType.DMA,
          pltpu.SemaphoreType.DMA,
          pltpu.SemaphoreType.DMA,
      ],
  )
  def sc_k(uidx_h, bidx_h, uemb_h, bemb_h, ubt_h, bbt_h,
           part_o, ub_o, bb_o,
           uidx_v, bidx_v,
           u0, u1, b0, b1, ubias_v, bbias_v, accv,
           sem_u0, sem_u1, sem_b0, sem_b1, sem_bias):
    ubufs, bbufs = (u0, u1), (b0, b1)
    usems, bsems = (sem_u0, sem_u1), (sem_b0, sem_b1)
    wid = lax.axis_index("s") * 2 + lax.axis_index("c")
    row0 = wid * _NCH
    pltpu.sync_copy(uidx_h.at[pl.ds(row0, _NCH)], uidx_v)
    pltpu.sync_copy(bidx_h.at[pl.ds(row0, _NCH)], bidx_v)

    bias_copies = []
    for j in range(_NCH):
      bias_copies.append(
          pltpu.async_copy(ubt_h.at[uidx_v.at[j]], ubias_v.at[j], sem_bias))
      bias_copies.append(
          pltpu.async_copy(bbt_h.at[bidx_v.at[j]], bbias_v.at[j], sem_bias))

    def issue_chunk(j):
      bu, bb2 = ubufs[j % 2], bbufs[j % 2]
      su, sb = usems[j % 2], bsems[j % 2]

      def it(g, _):
        base = g * 16
        u16 = uidx_v[j, pl.ds(base, 16)]
        b16 = bidx_v[j, pl.ds(base, 16)]
        for t in range(16):
          pltpu.async_copy(uemb_h.at[u16[t]], bu.at[base + t], su)
          pltpu.async_copy(bemb_h.at[b16[t]], bb2.at[base + t], sb)
        return 0

      lax.fori_loop(0, _CH // 16, it, 0)

    def drain_chunk(j):
      pltpu.make_async_copy(
          uemb_h.at[pl.ds(0, _CH)], ubufs[j % 2], usems[j % 2]).wait()
      pltpu.make_async_copy(
          bemb_h.at[pl.ds(0, _CH)], bbufs[j % 2], bsems[j % 2]).wait()

    issue_chunk(0)
    acc = jnp.zeros((16,), _f32)
    for j in range(_NCH):
      drain_chunk(j)
      if j + 1 < _NCH:
        issue_chunk(j + 1)
      bu, bb2 = ubufs[j % 2], bbufs[j % 2]

      def body(r, a, bu=bu, bb2=bb2):
        for k in range(_EMB // 16):
          sl = pl.ds(k * 16, 16)
          a = a + bu[r, sl] * bb2[r, sl]
        return a

      acc = lax.fori_loop(0, _CH, body, acc)

    accv[pl.ds(0, 16)] = acc
    for t in range(1, 8):
      accv[pl.ds(t * 16, 16)] = jnp.zeros((16,), _f32)
    for c in bias_copies:
      c.wait()

    pltpu.sync_copy(accv, part_o.at[wid])
    out_sl = pl.ds(row0, _NCH)
    pltpu.sync_copy(ubias_v, ub_o.at[out_sl])
    pltpu.sync_copy(bbias_v, bb_o.at[out_sl])

  return sc_k(uidx, bidx, uemb, bemb, ubt1, bbt1)


def _tc_body(part_ref, ub_ref, bb_ref, o_ref):
  s = jnp.sum(part_ref[...])
  o_ref[...] = jax.nn.sigmoid(ub_ref[...] + bb_ref[...] + s)


def kernel(inputs, user_embedding, user_bias_table, book_embedding,
           book_bias_table):
  idx = inputs.astype(jnp.int32)
  uidx = idx[:, 0].reshape(128, 128)
  bidx = idx[:, 1].reshape(128, 128)
  ubt1 = user_bias_table.reshape(100000)
  bbt1 = book_bias_table.reshape(100000)
  uscr, bscr = _sc_transpose(
      user_embedding.T, book_embedding.T,
      user_embedding[_NBLK * 128:, :], book_embedding[_NBLK * 128:, :])
  partials, ub, bb = _sc_main(uidx, bidx, uscr, bscr, ubt1, bbt1)
  out = pl.pallas_call(
      _tc_body,
      out_shape=jax.ShapeDtypeStruct((128, 128), _f32),
  )(partials, ub, bb)
  return out.reshape(_B, 1)


# speedup vs baseline: 3.4462x; 3.4462x over previous
import functools

import jax
import jax.numpy as jnp
from jax import lax
from jax.experimental import pallas as pl
from jax.experimental.pallas import tpu as pltpu
from jax.experimental.pallas import tpu_sc as plsc

_B = 16384
_EMB = 64
_NW = 32
_BPW = _B // _NW
_NCH = 4
_CH = 128

_f32 = jnp.float32


def _sc_bias(uidx, bidx, ubt1, bbt1):
  mesh = plsc.VectorSubcoreMesh(
      core_axis_name="c", subcore_axis_name="s", num_cores=2, num_subcores=16)

  @functools.partial(
      pl.kernel,
      out_type=(
          jax.ShapeDtypeStruct((128, 128), _f32),
          jax.ShapeDtypeStruct((128, 128), _f32),
      ),
      mesh=mesh,
      compiler_params=pltpu.CompilerParams(
          use_tc_tiling_on_sc=True, disable_bounds_checks=True),
      scratch_types=[
          pltpu.VMEM((_NCH, _CH), jnp.int32),
          pltpu.VMEM((_NCH, _CH), jnp.int32),
          pltpu.VMEM((_NCH, _CH), _f32),
          pltpu.VMEM((_NCH, _CH), _f32),
          pltpu.SemaphoreType.DMA,
      ],
  )
  def bias_k(uidx_h, bidx_h, ubt_h, bbt_h, ub_o, bb_o,
             uidx_v, bidx_v, ubias_v, bbias_v, sem_bias):
    wid = lax.axis_index("s") * 2 + lax.axis_index("c")
    row0 = wid * _NCH
    pltpu.sync_copy(uidx_h.at[pl.ds(row0, _NCH)], uidx_v)
    pltpu.sync_copy(bidx_h.at[pl.ds(row0, _NCH)], bidx_v)
    bias_copies = []
    for j in range(_NCH):
      bias_copies.append(
          pltpu.async_copy(ubt_h.at[uidx_v.at[j]], ubias_v.at[j], sem_bias))
      bias_copies.append(
          pltpu.async_copy(bbt_h.at[bidx_v.at[j]], bbias_v.at[j], sem_bias))
    for c in bias_copies:
      c.wait()
    out_sl = pl.ds(row0, _NCH)
    pltpu.sync_copy(ubias_v, ub_o.at[out_sl])
    pltpu.sync_copy(bbias_v, bb_o.at[out_sl])

  return bias_k(uidx, bidx, ubt1, bbt1)


def _sc_main(uidx, bidx, uemb, bemb):
  mesh = plsc.VectorSubcoreMesh(
      core_axis_name="c", subcore_axis_name="s", num_cores=2, num_subcores=16)

  @functools.partial(
      pl.kernel,
      out_type=jax.ShapeDtypeStruct((_NW, 128), _f32),
      mesh=mesh,
      compiler_params=pltpu.CompilerParams(
          use_tc_tiling_on_sc=True, disable_bounds_checks=True),
      scratch_types=[
          pltpu.VMEM((_NCH, _CH), jnp.int32),
          pltpu.VMEM((_NCH, _CH), jnp.int32),
          pltpu.VMEM((_CH, _EMB), _f32),
          pltpu.VMEM((_CH, _EMB), _f32),
          pltpu.VMEM((_CH, _EMB), _f32),
          pltpu.VMEM((_CH, _EMB), _f32),
          pltpu.VMEM((128,), _f32),
          pltpu.SemaphoreType.DMA,
          pltpu.SemaphoreType.DMA,
          pltpu.SemaphoreType.DMA,
          pltpu.SemaphoreType.DMA,
      ],
  )
  def sc_k(uidx_h, bidx_h, uemb_h, bemb_h,
           part_o,
           uidx_v, bidx_v,
           u0, u1, b0, b1, accv,
           sem_u0, sem_u1, sem_b0, sem_b1):
    ubufs, bbufs = (u0, u1), (b0, b1)
    usems, bsems = (sem_u0, sem_u1), (sem_b0, sem_b1)
    wid = lax.axis_index("s") * 2 + lax.axis_index("c")
    row0 = wid * _NCH
    pltpu.sync_copy(uidx_h.at[pl.ds(row0, _NCH)], uidx_v)
    pltpu.sync_copy(bidx_h.at[pl.ds(row0, _NCH)], bidx_v)

    def issue_chunk(j):
      bu, bb2 = ubufs[j % 2], bbufs[j % 2]
      su, sb = usems[j % 2], bsems[j % 2]

      def it(g, _):
        base = g * 16
        u16 = uidx_v[j, pl.ds(base, 16)]
        b16 = bidx_v[j, pl.ds(base, 16)]
        for t in range(16):
          pltpu.async_copy(uemb_h.at[u16[t]], bu.at[base + t], su)
          pltpu.async_copy(bemb_h.at[b16[t]], bb2.at[base + t], sb)
        return 0

      lax.fori_loop(0, _CH // 16, it, 0)

    def drain_chunk(j):
      pltpu.make_async_copy(
          uemb_h.at[pl.ds(0, _CH)], ubufs[j % 2], usems[j % 2]).wait()
      pltpu.make_async_copy(
          bemb_h.at[pl.ds(0, _CH)], bbufs[j % 2], bsems[j % 2]).wait()

    issue_chunk(0)
    acc = jnp.zeros((16,), _f32)
    for j in range(_NCH):
      drain_chunk(j)
      if j + 1 < _NCH:
        issue_chunk(j + 1)
      bu, bb2 = ubufs[j % 2], bbufs[j % 2]

      def body(r, a, bu=bu, bb2=bb2):
        for k in range(_EMB // 16):
          sl = pl.ds(k * 16, 16)
          a = a + bu[r, sl] * bb2[r, sl]
        return a

      acc = lax.fori_loop(0, _CH, body, acc)

    accv[pl.ds(0, 16)] = acc
    for t in range(1, 8):
      accv[pl.ds(t * 16, 16)] = jnp.zeros((16,), _f32)
    pltpu.sync_copy(accv, part_o.at[wid])

  return sc_k(uidx, bidx, uemb, bemb)


def _tc_body(part_ref, ub_ref, bb_ref, o_ref):
  s = jnp.sum(part_ref[...])
  o_ref[...] = jax.nn.sigmoid(ub_ref[...] + bb_ref[...] + s)


def kernel(inputs, user_embedding, user_bias_table, book_embedding,
           book_bias_table):
  idx = inputs.astype(jnp.int32)
  uidx = idx[:, 0].reshape(128, 128)
  bidx = idx[:, 1].reshape(128, 128)
  ubt1 = user_bias_table.reshape(100000)
  bbt1 = book_bias_table.reshape(100000)
  ub, bb = _sc_bias(uidx, bidx, ubt1, bbt1)
  partials = _sc_main(uidx, bidx, user_embedding, book_embedding)
  out = pl.pallas_call(
      _tc_body,
      out_shape=jax.ShapeDtypeStruct((128, 128), _f32),
  )(partials, ub, bb)
  return out.reshape(_B, 1)


# speedup vs baseline: 3.5149x; 1.0199x over previous
import functools

import jax
import jax.numpy as jnp
from jax import lax
from jax.experimental import pallas as pl
from jax.experimental.pallas import tpu as pltpu
from jax.experimental.pallas import tpu_sc as plsc

_B = 16384
_EMB = 64
_NW = 32
_BPW = _B // _NW
_NCH = 4
_CH = 128

_f32 = jnp.float32


def _sc_main(uidx, bidx, uemb, bemb, ubt1, bbt1):
  mesh = plsc.VectorSubcoreMesh(
      core_axis_name="c", subcore_axis_name="s", num_cores=2, num_subcores=16)

  @functools.partial(
      pl.kernel,
      out_type=(
          jax.ShapeDtypeStruct((_NW, 128), _f32),
          jax.ShapeDtypeStruct((128, 128), _f32),
          jax.ShapeDtypeStruct((128, 128), _f32),
      ),
      mesh=mesh,
      compiler_params=pltpu.CompilerParams(
          use_tc_tiling_on_sc=True, disable_bounds_checks=True),
      scratch_types=[
          pltpu.VMEM((_NCH, _CH), jnp.int32),
          pltpu.VMEM((_NCH, _CH), jnp.int32),
          pltpu.VMEM((_CH, _EMB), _f32),
          pltpu.VMEM((_CH, _EMB), _f32),
          pltpu.VMEM((_CH, _EMB), _f32),
          pltpu.VMEM((_CH, _EMB), _f32),
          pltpu.VMEM((_NCH, _CH), _f32),
          pltpu.VMEM((_NCH, _CH), _f32),
          pltpu.VMEM((128,), _f32),
          pltpu.SemaphoreType.DMA,
          pltpu.SemaphoreType.DMA,
          pltpu.SemaphoreType.DMA,
          pltpu.SemaphoreType.DMA,
          pltpu.SemaphoreType.DMA,
      ],
  )
  def sc_k(uidx_h, bidx_h, uemb_h, bemb_h, ubt_h, bbt_h,
           part_o, ub_o, bb_o,
           uidx_v, bidx_v,
           u0, u1, b0, b1, ubias_v, bbias_v, accv,
           sem_u0, sem_u1, sem_b0, sem_b1, sem_bias):
    ubufs, bbufs = (u0, u1), (b0, b1)
    usems, bsems = (sem_u0, sem_u1), (sem_b0, sem_b1)
    wid = lax.axis_index("s") * 2 + lax.axis_index("c")
    row0 = wid * _NCH
    pltpu.sync_copy(uidx_h.at[pl.ds(row0, _NCH)], uidx_v)
    pltpu.sync_copy(bidx_h.at[pl.ds(row0, _NCH)], bidx_v)

    bias_copies = []
    for j in range(_NCH):
      bias_copies.append(
          pltpu.async_copy(ubt_h.at[uidx_v.at[j]], ubias_v.at[j], sem_bias))
      bias_copies.append(
          pltpu.async_copy(bbt_h.at[bidx_v.at[j]], bbias_v.at[j], sem_bias))

    def issue_chunk(j):
      bu, bb2 = ubufs[j % 2], bbufs[j % 2]
      su, sb = usems[j % 2], bsems[j % 2]

      def it(g, _):
        base = g * 16
        u16 = uidx_v[j, pl.ds(base, 16)]
        b16 = bidx_v[j, pl.ds(base, 16)]
        for t in range(16):
          pltpu.async_copy(uemb_h.at[u16[t]], bu.at[base + t], su)
          pltpu.async_copy(bemb_h.at[b16[t]], bb2.at[base + t], sb)
        return 0

      lax.fori_loop(0, _CH // 16, it, 0)

    def drain_chunk(j):
      pltpu.make_async_copy(
          uemb_h.at[pl.ds(0, _CH)], ubufs[j % 2], usems[j % 2]).wait()
      pltpu.make_async_copy(
          bemb_h.at[pl.ds(0, _CH)], bbufs[j % 2], bsems[j % 2]).wait()

    issue_chunk(0)
    acc = jnp.zeros((16,), _f32)
    for j in range(_NCH):
      drain_chunk(j)
      if j + 1 < _NCH:
        issue_chunk(j + 1)
      bu, bb2 = ubufs[j % 2], bbufs[j % 2]

      def body(r, a, bu=bu, bb2=bb2):
        for k in range(_EMB // 16):
          sl = pl.ds(k * 16, 16)
          a = a + bu[r, sl] * bb2[r, sl]
        return a

      acc = lax.fori_loop(0, _CH, body, acc)

    accv[pl.ds(0, 16)] = acc
    for t in range(1, 8):
      accv[pl.ds(t * 16, 16)] = jnp.zeros((16,), _f32)
    for c in bias_copies:
      c.wait()

    pltpu.sync_copy(accv, part_o.at[wid])
    out_sl = pl.ds(row0, _NCH)
    pltpu.sync_copy(ubias_v, ub_o.at[out_sl])
    pltpu.sync_copy(bbias_v, bb_o.at[out_sl])

  return sc_k(uidx, bidx, uemb, bemb, ubt1, bbt1)


def _tc_body(part_ref, ub_ref, bb_ref, o_ref):
  s = jnp.sum(part_ref[...])
  o_ref[...] = jax.nn.sigmoid(ub_ref[...] + bb_ref[...] + s)


def kernel(inputs, user_embedding, user_bias_table, book_embedding,
           book_bias_table):
  idx = inputs.astype(jnp.int32)
  uidx = idx[:, 0].reshape(128, 128)
  bidx = idx[:, 1].reshape(128, 128)
  ubt1 = user_bias_table.reshape(100000)
  bbt1 = book_bias_table.reshape(100000)
  partials, ub, bb = _sc_main(uidx, bidx, user_embedding, book_embedding,
                              ubt1, bbt1)
  out = pl.pallas_call(
      _tc_body,
      out_shape=jax.ShapeDtypeStruct((128, 128), _f32),
  )(partials, ub, bb)
  return out.reshape(_B, 1)
